# Initial kernel scaffold; baseline (speedup 1.0000x reference)
import jax
import jax.numpy as jnp
from jax.experimental import pallas as pl


def kernel(nodes, history_uv, history_r, feat_table, r_table, W_gv, b_gv, W1, b1):
    raise NotImplementedError("write your pallas kernel here")


# speedup vs baseline: 1.0577x; 1.0577x over previous
import functools

import jax
import jax.numpy as jnp
from jax import lax
from jax.experimental import pallas as pl
from jax.experimental.pallas import tpu as pltpu
from jax.experimental.pallas import tpu_sc as plsc

B = 16384
L = 32
D = 128
N_NODES = 100000
NW = 32
ROWS_PER_W = B // NW
CH_ROWS = 4
CH = CH_ROWS * L
N_CHUNKS = ROWS_PER_W // CH_ROWS


def _proj_body(feat_ref, wg_ref, ws_ref, outg_ref, outs_ref):
    x = feat_ref[...]
    outg_ref[...] = jnp.dot(x, wg_ref[...], preferred_element_type=jnp.float32)
    outs_ref[...] = jnp.dot(x, ws_ref[...], preferred_element_type=jnp.float32)


def _rproj_body(rt_ref, wgb_ref, bgv_ref, out_ref):
    out_ref[...] = (
        jnp.dot(rt_ref[...], wgb_ref[...], preferred_element_type=jnp.float32)
        + bgv_ref[...]
    )


def _final_body(selfc_ref, neigh_ref, w1b_ref, b1_ref, out_ref):
    neigh = neigh_ref[...] * (1.0 / L)
    y = (
        selfc_ref[...]
        + jnp.dot(neigh, w1b_ref[...], preferred_element_type=jnp.float32)
        + b1_ref[...]
    )
    out_ref[...] = jnp.maximum(y, 0.0)


def _sc_body(uv_hbm, r_hbm, nodes_hbm, projg_hbm, rproj_hbm, projs_hbm,
             neigh_hbm, selfc_hbm,
             uvidx_v, ridx_v, fbuf, rbuf, obuf, sem1, sem2):
    wid = lax.axis_index("s") * 2 + lax.axis_index("c")
    row0 = wid * ROWS_PER_W

    def chunk_body(c, carry):
        base_row = row0 + c * CH_ROWS
        pair0 = base_row * L
        pltpu.sync_copy(uv_hbm.at[pl.ds(pair0, CH)], uvidx_v)
        pltpu.sync_copy(r_hbm.at[pl.ds(pair0, CH)], ridx_v)
        cp1 = pltpu.async_copy(projg_hbm.at[uvidx_v], fbuf, sem1)
        cp2 = pltpu.async_copy(rproj_hbm.at[ridx_v], rbuf, sem2)
        cp1.wait()
        cp2.wait()
        for g in range(CH_ROWS):
            def pair_body(p, acc):
                row = g * L + p
                return tuple(
                    acc[j]
                    + jnp.maximum(
                        fbuf[row, pl.ds(16 * j, 16)]
                        + rbuf[row, pl.ds(16 * j, 16)],
                        0.0,
                    )
                    for j in range(8)
                )
            acc0 = tuple(jnp.zeros((16,), jnp.float32) for _ in range(8))
            acc = lax.fori_loop(0, L, pair_body, acc0)
            for j in range(8):
                obuf[g, pl.ds(16 * j, 16)] = acc[j]
        pltpu.sync_copy(obuf, neigh_hbm.at[pl.ds(base_row, CH_ROWS)])
        return carry

    lax.fori_loop(0, N_CHUNKS, chunk_body, 0)

    def self_body(c, carry):
        base_row = row0 + c * CH
        pltpu.sync_copy(nodes_hbm.at[pl.ds(base_row, CH)], uvidx_v)
        pltpu.async_copy(projs_hbm.at[uvidx_v], fbuf, sem1).wait()
        pltpu.sync_copy(fbuf, selfc_hbm.at[pl.ds(base_row, CH)])
        return carry

    lax.fori_loop(0, ROWS_PER_W // CH, self_body, 0)


_sc_gather = functools.partial(
    pl.kernel,
    out_type=(
        jax.ShapeDtypeStruct((B, D), jnp.float32),
        jax.ShapeDtypeStruct((B, D), jnp.float32),
    ),
    mesh=plsc.VectorSubcoreMesh(core_axis_name="c", subcore_axis_name="s"),
    scratch_types=[
        pltpu.VMEM((CH,), jnp.int32),
        pltpu.VMEM((CH,), jnp.int32),
        pltpu.VMEM((CH, D), jnp.float32),
        pltpu.VMEM((CH, D), jnp.float32),
        pltpu.VMEM((CH_ROWS, D), jnp.float32),
        pltpu.SemaphoreType.DMA,
        pltpu.SemaphoreType.DMA,
    ],
)(_sc_body)


def kernel(nodes, history_uv, history_r, feat_table, r_table, W_gv, b_gv, W1, b1):
    nodes = nodes.astype(jnp.int32)
    uv = history_uv.astype(jnp.int32).reshape(-1)
    hr = history_r.astype(jnp.int32).reshape(-1)

    wg_a, wg_b = W_gv[:D], W_gv[D:]
    w1_a, w1_b = W1[:D], W1[D:]

    rb = 512
    n_rblocks = pl.cdiv(N_NODES, rb)
    projg, projs = pl.pallas_call(
        _proj_body,
        grid=(n_rblocks,),
        in_specs=[
            pl.BlockSpec((rb, D), lambda i: (i, 0)),
            pl.BlockSpec((D, D), lambda i: (0, 0)),
            pl.BlockSpec((D, D), lambda i: (0, 0)),
        ],
        out_specs=[
            pl.BlockSpec((rb, D), lambda i: (i, 0)),
            pl.BlockSpec((rb, D), lambda i: (i, 0)),
        ],
        out_shape=(
            jax.ShapeDtypeStruct((N_NODES, D), jnp.float32),
            jax.ShapeDtypeStruct((N_NODES, D), jnp.float32),
        ),
    )(feat_table, wg_a, w1_a)

    r_pad = jnp.zeros((8, D), jnp.float32).at[:6].set(r_table)
    rproj = pl.pallas_call(
        _rproj_body,
        out_shape=jax.ShapeDtypeStruct((8, D), jnp.float32),
    )(r_pad, wg_b, b_gv.reshape(1, D))

    neigh_sum, selfc = _sc_gather(uv, hr, nodes, projg, rproj, projs)

    bb = 1024
    out = pl.pallas_call(
        _final_body,
        grid=(B // bb,),
        in_specs=[
            pl.BlockSpec((bb, D), lambda i: (i, 0)),
            pl.BlockSpec((bb, D), lambda i: (i, 0)),
            pl.BlockSpec((D, D), lambda i: (0, 0)),
            pl.BlockSpec((1, D), lambda i: (0, 0)),
        ],
        out_specs=pl.BlockSpec((bb, D), lambda i: (i, 0)),
        out_shape=jax.ShapeDtypeStruct((B, D), jnp.float32),
    )(selfc, neigh_sum, w1_b, b1.reshape(1, D))
    return out
